# baseline (speedup 1.0000x reference)
import jax
import jax.numpy as jnp
from jax.experimental import pallas as pl
from jax.experimental.pallas import tpu as pltpu

D_KQ = 128
D_VAL = 128
N_MEM = 100000
BQ, TQ = 8, 64
K_TOP = 32
TEMP = 0.1

BLK_M = 2048
N_MEM_PAD = 100352
N_BLOCKS = N_MEM_PAD // BLK_M


def _scores_body(nq_ref, keys_ref, out_ref):
    j = pl.program_id(0)
    k = keys_ref[...]
    norm = jnp.sqrt(jnp.sum(k * k, axis=-1, keepdims=True))
    nk = k / jnp.maximum(norm, 1e-12)
    s = jax.lax.dot_general(
        nq_ref[...], nk,
        dimension_numbers=(((1,), (1,)), ((), ())),
        preferred_element_type=jnp.float32,
    )
    col = j * BLK_M + jax.lax.broadcasted_iota(jnp.int32, s.shape, 1)
    out_ref[...] = jnp.where(col < N_MEM, s, -2.0)


def _scores(nq, keys_pad):
    return pl.pallas_call(
        _scores_body,
        grid=(N_BLOCKS,),
        in_specs=[
            pl.BlockSpec((BQ * TQ, D_KQ), lambda j: (0, 0)),
            pl.BlockSpec((BLK_M, D_KQ), lambda j: (j, 0)),
        ],
        out_specs=pl.BlockSpec((BQ * TQ, BLK_M), lambda j: (0, j)),
        out_shape=jax.ShapeDtypeStruct((BQ * TQ, N_MEM_PAD), jnp.float32),
        compiler_params=pltpu.CompilerParams(
            dimension_semantics=("arbitrary",),
        ),
    )(nq, keys_pad)


def kernel(queries, keys, values):
    b, t, d_k = queries.shape
    flat_q = queries.reshape(b * t, d_k)
    nq = flat_q / jnp.maximum(
        jnp.linalg.norm(flat_q, axis=-1, keepdims=True), 1e-12)
    keys_pad = jnp.pad(keys, ((0, N_MEM_PAD - N_MEM), (0, 0)))
    scores = _scores(nq, keys_pad)
    topk_scores, topk_idx = jax.lax.top_k(scores, K_TOP)
    attn = jax.nn.softmax(topk_scores / TEMP, axis=-1)
    retrieved = values[topk_idx]
    agg = jnp.sum(attn[..., None] * retrieved, axis=-2)
    return (agg.reshape(b, t, D_VAL),
            attn.reshape(b, t, K_TOP),
            topk_idx.reshape(b, t, K_TOP))


# speedup vs baseline: 2.6160x; 2.6160x over previous
import functools

import jax
import jax.numpy as jnp
from jax import lax
from jax.experimental import pallas as pl
from jax.experimental.pallas import tpu as pltpu
from jax.experimental.pallas import tpu_sc as plsc

D_KQ = 128
D_VAL = 128
N_MEM = 100000
BQ, TQ = 8, 64
NQROWS = BQ * TQ
K_TOP = 32
TEMP = 0.1

BLK_M = 2048
N_MEM_PAD = 100352
N_BLOCKS = N_MEM_PAD // BLK_M
N_PAR = 8
G_LANES = 128
N_GROUPS = N_PAR * G_LANES
N_SEL = 48
N_A = 7
MEMB = N_A * 16
CWIDTH = N_SEL * MEMB
PAD_COL = N_MEM_PAD - 1

NC, NS, L = 2, 16, 16
NW = NC * NS
ROWS_PER_W = NQROWS // NW
IDX_CHUNK = 128
N_IDX_CHUNKS = CWIDTH // IDX_CHUNK


def _k1_body(nq_ref, keys_ref, scores_ref, gmax_ref):
    j = pl.program_id(0)
    nk = keys_ref[...]
    s = jax.lax.dot_general(
        nq_ref[...], nk,
        dimension_numbers=(((1,), (1,)), ((), ())),
        preferred_element_type=jnp.float32,
    )
    col = j * BLK_M + jax.lax.broadcasted_iota(jnp.int32, s.shape, 1)
    s = jnp.where(col < N_MEM, s, -2.0)
    scores_ref[...] = s
    bmax = jnp.max(s.reshape(NQROWS, BLK_M // G_LANES, G_LANES), axis=1)
    half = (j % N_PAR) * G_LANES
    old = gmax_ref[:, pl.ds(half, G_LANES)]
    acc = jnp.where(j < N_PAR, bmax, jnp.maximum(old, bmax))
    gmax_ref[:, pl.ds(half, G_LANES)] = acc


def _k1(nq, keys_pad):
    return pl.pallas_call(
        _k1_body,
        grid=(N_BLOCKS,),
        in_specs=[
            pl.BlockSpec((NQROWS, D_KQ), lambda j: (0, 0)),
            pl.BlockSpec((BLK_M, D_KQ), lambda j: (j, 0)),
        ],
        out_specs=[
            pl.BlockSpec((NQROWS, BLK_M), lambda j: (0, j)),
            pl.BlockSpec((NQROWS, N_GROUPS), lambda j: (0, 0)),
        ],
        out_shape=[
            jax.ShapeDtypeStruct((NQROWS, N_MEM_PAD), jnp.float32),
            jax.ShapeDtypeStruct((NQROWS, N_GROUPS), jnp.float32),
        ],
        compiler_params=pltpu.CompilerParams(
            dimension_semantics=("arbitrary",),
        ),
    )(nq, keys_pad)


@functools.lru_cache(maxsize=1)
def _k3_make():
    mesh = plsc.VectorSubcoreMesh(core_axis_name="c", subcore_axis_name="s")

    @functools.partial(
        pl.kernel,
        mesh=mesh,
        out_type=jax.ShapeDtypeStruct((NQROWS, CWIDTH), jnp.float32),
        scratch_types=[
            pltpu.VMEM((CWIDTH,), jnp.int32),
            pltpu.VMEM((CWIDTH,), jnp.float32),
            pltpu.SemaphoreType.DMA,
        ],
    )
    def k3(scores_hbm, cols_hbm, cand_out, idxb, candb, sem):
        wid = lax.axis_index("s") * NC + lax.axis_index("c")

        def row_body(r, _):
            row = wid * ROWS_PER_W + r
            pltpu.sync_copy(cols_hbm.at[row], idxb)
            handles = []
            for k in range(N_IDX_CHUNKS):
                sl = pl.ds(k * IDX_CHUNK, IDX_CHUNK)
                handles.append(pltpu.async_copy(
                    scores_hbm.at[idxb.at[sl]], candb.at[sl], sem))
            for h in handles:
                h.wait()
            pltpu.sync_copy(candb, cand_out.at[row])
            return 0

        lax.fori_loop(0, ROWS_PER_W, row_body, 0)

    return k3


def _k3(scores_flat, colsmat):
    return _k3_make()(scores_flat, colsmat)


def kernel(queries, keys, values):
    b, t, d_k = queries.shape
    flat_q = queries.reshape(b * t, d_k)
    nq = flat_q / jnp.maximum(
        jnp.linalg.norm(flat_q, axis=-1, keepdims=True), 1e-12)
    nk = keys / jnp.maximum(
        jnp.linalg.norm(keys, axis=-1, keepdims=True), 1e-12)
    keys_pad = jnp.pad(nk, ((0, N_MEM_PAD - N_MEM), (0, 0)))
    scores, gmax = _k1(nq, keys_pad)

    _, gid = jax.lax.top_k(gmax, N_SEL)

    p = gid // G_LANES
    c = gid % G_LANES
    a = jnp.arange(N_A, dtype=jnp.int32)[:, None]
    i = jnp.arange(16, dtype=jnp.int32)[None, :]
    off = (N_PAR * BLK_M) * a + G_LANES * i
    col = (BLK_M * p + c)[..., None, None] + off[None, None]
    valid = (p[..., None, None] == 0) | (a[None, None] < N_A - 1)
    col = jnp.where(valid, col, PAD_COL)
    rowbase = jnp.arange(NQROWS, dtype=jnp.int32)[:, None] * N_MEM_PAD
    colsmat = (col.reshape(NQROWS, CWIDTH) + rowbase).astype(jnp.int32)

    scores_flat = scores.reshape(NQROWS * N_MEM_PAD)
    cand = _k3(scores_flat, colsmat)

    N_PRE = 40
    pre_vals, pos = jax.lax.top_k(cand, N_PRE)
    slot = pos // MEMB
    m = pos % MEMB
    a_s = m // 16
    i_s = m % 16
    sel = jnp.arange(N_SEL, dtype=jnp.int32)[None, None, :] == slot[..., None]
    gsel = jnp.sum(jnp.where(sel, gid[:, None, :], 0), axis=-1)
    pre_idx = ((N_PAR * BLK_M) * a_s + BLK_M * (gsel // G_LANES)
               + G_LANES * i_s + gsel % G_LANES)
    vi = pre_vals[:, :, None]
    vj = pre_vals[:, None, :]
    ci = pre_idx[:, :, None]
    cj = pre_idx[:, None, :]
    beats = (vj > vi) | ((vj == vi) & (cj < ci))
    rank = jnp.sum(beats.astype(jnp.int32), axis=-1)
    oh = rank[:, None, :] == jnp.arange(K_TOP,
                                        dtype=jnp.int32)[None, :, None]
    top_vals = jnp.sum(jnp.where(oh, vi.transpose(0, 2, 1), 0.0), axis=-1)
    top_idx = jnp.sum(jnp.where(oh, ci.transpose(0, 2, 1), 0), axis=-1)

    attn = jax.nn.softmax(top_vals / TEMP, axis=-1)
    retrieved = values[top_idx]
    agg = jnp.sum(attn[..., None] * retrieved, axis=-2)
    return (agg.reshape(b, t, D_VAL),
            attn.reshape(b, t, K_TOP),
            top_idx.reshape(b, t, K_TOP))
